# baseline (speedup 1.0000x reference)
import functools

import jax
import jax.numpy as jnp
from jax import lax
from jax.experimental import pallas as pl
from jax.experimental.pallas import tpu as pltpu
from jax.experimental.pallas import tpu_sc as plsc

_NUM_CORES = 2
_NUM_SUBCORES = 16
_NW = _NUM_CORES * _NUM_SUBCORES
_LANES = 16
_CHUNK = 128


def _sc_scores_body(ids_hbm, attn_hbm, tfidf_hbm, sp0_h, sp1_h, sp2_h, sp3_h,
                    dflt_h, out_hbm, idx_v, attn_v, vals_v, sp0_v, sp1_v,
                    sp2_v, sp3_v, dflt_v, sem):
    n_tok = ids_hbm.shape[0]
    per_w = n_tok // _NW
    n_chunks = per_w // _CHUNK
    wid = lax.axis_index("s") * _NUM_CORES + lax.axis_index("c")
    base = wid * per_w

    pltpu.sync_copy(ids_hbm.at[pl.ds(base, per_w)], idx_v)
    pltpu.sync_copy(attn_hbm.at[pl.ds(base, per_w)], attn_v)
    pltpu.sync_copy(sp0_h, sp0_v)
    pltpu.sync_copy(sp1_h, sp1_v)
    pltpu.sync_copy(sp2_h, sp2_v)
    pltpu.sync_copy(sp3_h, sp3_v)
    pltpu.sync_copy(dflt_h, dflt_v)

    descs = [
        pltpu.async_copy(
            tfidf_hbm.at[idx_v.at[pl.ds(j * _CHUNK, _CHUNK)]],
            vals_v.at[pl.ds(j * _CHUNK, _CHUNK)],
            sem,
        )
        for j in range(n_chunks)
    ]
    for d in descs:
        d.wait()

    s0 = sp0_v[...]
    s1 = sp1_v[...]
    s2 = sp2_v[...]
    s3 = sp3_v[...]
    dflt = dflt_v[...]
    for t in range(per_w // _LANES):
        sl = pl.ds(t * _LANES, _LANES)
        tok = idx_v[sl]
        val = vals_v[sl]
        amask = attn_v[sl]
        is_special = (tok == s0) | (tok == s1) | (tok == s2) | (tok == s3)
        vals_v[sl] = jnp.where(is_special, dflt, val) * amask

    pltpu.sync_copy(vals_v, out_hbm.at[pl.ds(base, per_w)])


def _sc_scores(ids_flat, attn_flat, tfidf, specials, dflt_vec):
    n_tok = ids_flat.shape[0]
    per_w = n_tok // _NW
    call = functools.partial(
        pl.kernel,
        mesh=plsc.VectorSubcoreMesh(core_axis_name="c", subcore_axis_name="s"),
        out_type=jax.ShapeDtypeStruct((n_tok,), jnp.float32),
        scratch_types=[
            pltpu.VMEM((per_w,), jnp.int32),
            pltpu.VMEM((per_w,), jnp.float32),
            pltpu.VMEM((per_w,), jnp.float32),
            pltpu.VMEM((_LANES,), jnp.int32),
            pltpu.VMEM((_LANES,), jnp.int32),
            pltpu.VMEM((_LANES,), jnp.int32),
            pltpu.VMEM((_LANES,), jnp.int32),
            pltpu.VMEM((_LANES,), jnp.float32),
            pltpu.SemaphoreType.DMA,
        ],
    )(_sc_scores_body)
    return call(ids_flat, attn_flat, tfidf, specials[0], specials[1],
                specials[2], specials[3], dflt_vec)


def _tc_mul_body(n_blocks, emb_ref, sc_ref, tt_ref, attn_ref, out_ref,
                 mean_ref, ctx_ref, cmt_ref, acc):
    i = pl.program_id(0)

    @pl.when(i == 0)
    def _init():
        for k in range(6):
            acc[k] = 0.0

    sv = sc_ref[...]
    out_ref[...] = emb_ref[...] * sv
    amask = attn_ref[...]
    ctx = (tt_ref[...] == 1).astype(jnp.float32)
    cmt = (tt_ref[...] == 0).astype(jnp.float32)
    acc[0] = acc[0] + jnp.sum(sv)
    acc[1] = acc[1] + jnp.sum(amask)
    acc[2] = acc[2] + jnp.sum(sv * ctx)
    acc[3] = acc[3] + jnp.sum(ctx * amask)
    acc[4] = acc[4] + jnp.sum(sv * cmt)
    acc[5] = acc[5] + jnp.sum(cmt * amask)

    @pl.when(i == n_blocks - 1)
    def _final():
        mean_ref[...] = jnp.full((1, 1), acc[0] / (acc[1] + 1e-8), jnp.float32)
        ctx_ref[...] = jnp.full((1, 1), acc[2] / (acc[3] + 1e-8), jnp.float32)
        cmt_ref[...] = jnp.full((1, 1), acc[4] / (acc[5] + 1e-8), jnp.float32)


def _tc_mul(emb2d, scores2d, tt2d, attn2d, blk):
    n, d = emb2d.shape
    n_blocks = n // blk
    row_spec = pl.BlockSpec((blk, 1), lambda i: (i, 0))
    scalar_spec = pl.BlockSpec((1, 1), lambda i: (0, 0))
    return pl.pallas_call(
        functools.partial(_tc_mul_body, n_blocks),
        grid=(n_blocks,),
        in_specs=[
            pl.BlockSpec((blk, d), lambda i: (i, 0)),
            row_spec,
            row_spec,
            row_spec,
        ],
        out_specs=[
            pl.BlockSpec((blk, d), lambda i: (i, 0)),
            scalar_spec,
            scalar_spec,
            scalar_spec,
        ],
        out_shape=[
            jax.ShapeDtypeStruct((n, d), jnp.float32),
            jax.ShapeDtypeStruct((1, 1), jnp.float32),
            jax.ShapeDtypeStruct((1, 1), jnp.float32),
            jax.ShapeDtypeStruct((1, 1), jnp.float32),
        ],
        scratch_shapes=[pltpu.SMEM((8,), jnp.float32)],
    )(emb2d, scores2d, tt2d, attn2d)


def kernel(embeddings, input_ids, token_type_ids, attention_mask,
           special_token_ids, tfidf_scores, default_score):
    b, l, d = embeddings.shape
    n = b * l

    ids_flat = input_ids.reshape(n).astype(jnp.int32)
    attn_flat = attention_mask.reshape(n).astype(jnp.float32)
    sp = special_token_ids.astype(jnp.int32)
    specials = [jnp.full((_LANES,), sp[k], jnp.int32) for k in range(4)]
    dflt_vec = jnp.full((_LANES,), default_score, jnp.float32)

    scores = _sc_scores(ids_flat, attn_flat,
                        tfidf_scores.astype(jnp.float32), specials, dflt_vec)

    emb2d = embeddings.reshape(n, d)
    sc2d = scores.reshape(n, 1)
    tt2d = token_type_ids.reshape(n, 1).astype(jnp.int32)
    attn2d = attn_flat.reshape(n, 1)

    masked, mean_v, ctx_v, cmt_v = _tc_mul(emb2d, sc2d, tt2d, attn2d, blk=512)

    return (
        masked.reshape(b, l, d),
        scores.reshape(b, l, 1),
        mean_v[0, 0],
        ctx_v[0, 0],
        cmt_v[0, 0],
    )


# speedup vs baseline: 1.0460x; 1.0460x over previous
import functools

import jax
import jax.numpy as jnp
from jax import lax
from jax.experimental import pallas as pl
from jax.experimental.pallas import tpu as pltpu
from jax.experimental.pallas import tpu_sc as plsc

_NUM_CORES = 2
_NUM_SUBCORES = 16
_NW = _NUM_CORES * _NUM_SUBCORES
_LANES = 16
_CHUNK = 128


def _sc_scores_body(ids_hbm, attn_hbm, tfidf_hbm, sp0_h, sp1_h, sp2_h, sp3_h,
                    dflt_h, out_hbm, idx_v, attn_v, vals_v, sp0_v, sp1_v,
                    sp2_v, sp3_v, dflt_v, sem):
    n_tok = ids_hbm.shape[0]
    per_w = n_tok // _NW
    n_chunks = per_w // _CHUNK
    wid = lax.axis_index("s") * _NUM_CORES + lax.axis_index("c")
    base = wid * per_w

    pltpu.sync_copy(ids_hbm.at[pl.ds(base, per_w)], idx_v)
    pltpu.sync_copy(attn_hbm.at[pl.ds(base, per_w)], attn_v)
    pltpu.sync_copy(sp0_h, sp0_v)
    pltpu.sync_copy(sp1_h, sp1_v)
    pltpu.sync_copy(sp2_h, sp2_v)
    pltpu.sync_copy(sp3_h, sp3_v)
    pltpu.sync_copy(dflt_h, dflt_v)

    descs = [
        pltpu.async_copy(
            tfidf_hbm.at[idx_v.at[pl.ds(j * _CHUNK, _CHUNK)]],
            vals_v.at[pl.ds(j * _CHUNK, _CHUNK)],
            sem,
        )
        for j in range(n_chunks)
    ]
    for d in descs:
        d.wait()

    s0 = sp0_v[...]
    s1 = sp1_v[...]
    s2 = sp2_v[...]
    s3 = sp3_v[...]
    dflt = dflt_v[...]
    for t in range(per_w // _LANES):
        sl = pl.ds(t * _LANES, _LANES)
        tok = idx_v[sl]
        val = vals_v[sl]
        amask = attn_v[sl]
        is_special = (tok == s0) | (tok == s1) | (tok == s2) | (tok == s3)
        vals_v[sl] = jnp.where(is_special, dflt, val) * amask

    pltpu.sync_copy(vals_v, out_hbm.at[pl.ds(base, per_w)])


def _sc_scores(ids_flat, attn_flat, tfidf, specials, dflt_vec):
    n_tok = ids_flat.shape[0]
    per_w = n_tok // _NW
    call = functools.partial(
        pl.kernel,
        mesh=plsc.VectorSubcoreMesh(core_axis_name="c", subcore_axis_name="s"),
        out_type=jax.ShapeDtypeStruct((n_tok,), jnp.float32),
        scratch_types=[
            pltpu.VMEM((per_w,), jnp.int32),
            pltpu.VMEM((per_w,), jnp.float32),
            pltpu.VMEM((per_w,), jnp.float32),
            pltpu.VMEM((_LANES,), jnp.int32),
            pltpu.VMEM((_LANES,), jnp.int32),
            pltpu.VMEM((_LANES,), jnp.int32),
            pltpu.VMEM((_LANES,), jnp.int32),
            pltpu.VMEM((_LANES,), jnp.float32),
            pltpu.SemaphoreType.DMA,
        ],
    )(_sc_scores_body)
    return call(ids_flat, attn_flat, tfidf, specials[0], specials[1],
                specials[2], specials[3], dflt_vec)


def _tc_mul_body(n_blocks, emb_ref, sc_ref, tt_ref, attn_ref, out_ref,
                 mean_ref, ctx_ref, cmt_ref, acc):
    i = pl.program_id(0)

    @pl.when(i == 0)
    def _init():
        for k in range(6):
            acc[k] = 0.0

    sv = sc_ref[...]
    out_ref[...] = emb_ref[...] * sv
    amask = attn_ref[...]
    ctx = (tt_ref[...] == 1).astype(jnp.float32)
    cmt = (tt_ref[...] == 0).astype(jnp.float32)
    acc[0] = acc[0] + jnp.sum(sv)
    acc[1] = acc[1] + jnp.sum(amask)
    acc[2] = acc[2] + jnp.sum(sv * ctx)
    acc[3] = acc[3] + jnp.sum(ctx * amask)
    acc[4] = acc[4] + jnp.sum(sv * cmt)
    acc[5] = acc[5] + jnp.sum(cmt * amask)

    @pl.when(i == n_blocks - 1)
    def _final():
        mean_ref[...] = jnp.full((1, 1), acc[0] / (acc[1] + 1e-8), jnp.float32)
        ctx_ref[...] = jnp.full((1, 1), acc[2] / (acc[3] + 1e-8), jnp.float32)
        cmt_ref[...] = jnp.full((1, 1), acc[4] / (acc[5] + 1e-8), jnp.float32)


def _tc_mul(emb2d, scores2d, tt2d, attn2d, blk):
    n, d = emb2d.shape
    n_blocks = n // blk
    row_spec = pl.BlockSpec((blk, 1), lambda i: (i, 0))
    scalar_spec = pl.BlockSpec((1, 1), lambda i: (0, 0))
    return pl.pallas_call(
        functools.partial(_tc_mul_body, n_blocks),
        grid=(n_blocks,),
        in_specs=[
            pl.BlockSpec((blk, d), lambda i: (i, 0)),
            row_spec,
            row_spec,
            row_spec,
        ],
        out_specs=[
            pl.BlockSpec((blk, d), lambda i: (i, 0)),
            scalar_spec,
            scalar_spec,
            scalar_spec,
        ],
        out_shape=[
            jax.ShapeDtypeStruct((n, d), jnp.float32),
            jax.ShapeDtypeStruct((1, 1), jnp.float32),
            jax.ShapeDtypeStruct((1, 1), jnp.float32),
            jax.ShapeDtypeStruct((1, 1), jnp.float32),
        ],
        scratch_shapes=[pltpu.SMEM((8,), jnp.float32)],
    )(emb2d, scores2d, tt2d, attn2d)


def kernel(embeddings, input_ids, token_type_ids, attention_mask,
           special_token_ids, tfidf_scores, default_score):
    b, l, d = embeddings.shape
    n = b * l

    ids_flat = input_ids.reshape(n).astype(jnp.int32)
    attn_flat = attention_mask.reshape(n).astype(jnp.float32)
    sp = special_token_ids.astype(jnp.int32)
    specials = [jnp.full((_LANES,), sp[k], jnp.int32) for k in range(4)]
    dflt_vec = jnp.full((_LANES,), default_score, jnp.float32)

    scores = _sc_scores(ids_flat, attn_flat,
                        tfidf_scores.astype(jnp.float32), specials, dflt_vec)

    emb2d = embeddings.reshape(n, d)
    sc2d = scores.reshape(n, 1)
    tt2d = token_type_ids.reshape(n, 1).astype(jnp.int32)
    attn2d = attn_flat.reshape(n, 1)

    masked, mean_v, ctx_v, cmt_v = _tc_mul(emb2d, sc2d, tt2d, attn2d, blk=1024)

    return (
        masked.reshape(b, l, d),
        scores.reshape(b, l, 1),
        mean_v[0, 0],
        ctx_v[0, 0],
        cmt_v[0, 0],
    )


# speedup vs baseline: 1.0572x; 1.0108x over previous
import functools

import jax
import jax.numpy as jnp
from jax import lax
from jax.experimental import pallas as pl
from jax.experimental.pallas import tpu as pltpu
from jax.experimental.pallas import tpu_sc as plsc

_NUM_CORES = 2
_NUM_SUBCORES = 16
_NW = _NUM_CORES * _NUM_SUBCORES
_LANES = 16
_CHUNK = 128


def _sc_scores_body(ids_hbm, attn_hbm, tfidf_hbm, sp0_h, sp1_h, sp2_h, sp3_h,
                    dflt_h, out_hbm, idx_v, attn_v, vals_v, sp0_v, sp1_v,
                    sp2_v, sp3_v, dflt_v, sem):
    n_tok = ids_hbm.shape[0]
    per_w = n_tok // _NW
    n_chunks = per_w // _CHUNK
    wid = lax.axis_index("s") * _NUM_CORES + lax.axis_index("c")
    base = wid * per_w

    pltpu.sync_copy(ids_hbm.at[pl.ds(base, per_w)], idx_v)
    pltpu.sync_copy(attn_hbm.at[pl.ds(base, per_w)], attn_v)
    pltpu.sync_copy(sp0_h, sp0_v)
    pltpu.sync_copy(sp1_h, sp1_v)
    pltpu.sync_copy(sp2_h, sp2_v)
    pltpu.sync_copy(sp3_h, sp3_v)
    pltpu.sync_copy(dflt_h, dflt_v)

    descs = [
        pltpu.async_copy(
            tfidf_hbm.at[idx_v.at[pl.ds(j * _CHUNK, _CHUNK)]],
            vals_v.at[pl.ds(j * _CHUNK, _CHUNK)],
            sem,
        )
        for j in range(n_chunks)
    ]
    for d in descs:
        d.wait()

    s0 = sp0_v[...]
    s1 = sp1_v[...]
    s2 = sp2_v[...]
    s3 = sp3_v[...]
    dflt = dflt_v[...]
    for t in range(per_w // _LANES):
        sl = pl.ds(t * _LANES, _LANES)
        tok = idx_v[sl]
        val = vals_v[sl]
        amask = attn_v[sl]
        is_special = (tok == s0) | (tok == s1) | (tok == s2) | (tok == s3)
        vals_v[sl] = jnp.where(is_special, dflt, val) * amask

    pltpu.sync_copy(vals_v, out_hbm.at[pl.ds(base, per_w)])


def _sc_scores(ids_flat, attn_flat, tfidf, specials, dflt_vec):
    n_tok = ids_flat.shape[0]
    per_w = n_tok // _NW
    call = functools.partial(
        pl.kernel,
        mesh=plsc.VectorSubcoreMesh(core_axis_name="c", subcore_axis_name="s"),
        out_type=jax.ShapeDtypeStruct((n_tok,), jnp.float32),
        scratch_types=[
            pltpu.VMEM((per_w,), jnp.int32),
            pltpu.VMEM((per_w,), jnp.float32),
            pltpu.VMEM((per_w,), jnp.float32),
            pltpu.VMEM((_LANES,), jnp.int32),
            pltpu.VMEM((_LANES,), jnp.int32),
            pltpu.VMEM((_LANES,), jnp.int32),
            pltpu.VMEM((_LANES,), jnp.int32),
            pltpu.VMEM((_LANES,), jnp.float32),
            pltpu.SemaphoreType.DMA,
        ],
    )(_sc_scores_body)
    return call(ids_flat, attn_flat, tfidf, specials[0], specials[1],
                specials[2], specials[3], dflt_vec)


def _tc_mul_body(n_blocks, emb_ref, sc_ref, tt_ref, attn_ref, out_ref,
                 mean_ref, ctx_ref, cmt_ref, acc):
    i = pl.program_id(0)

    @pl.when(i == 0)
    def _init():
        for k in range(6):
            acc[k] = 0.0

    sv = sc_ref[...]
    out_ref[...] = emb_ref[...] * sv
    amask = attn_ref[...]
    ctx = (tt_ref[...] == 1).astype(jnp.float32)
    cmt = (tt_ref[...] == 0).astype(jnp.float32)
    acc[0] = acc[0] + jnp.sum(sv)
    acc[1] = acc[1] + jnp.sum(amask)
    acc[2] = acc[2] + jnp.sum(sv * ctx)
    acc[3] = acc[3] + jnp.sum(ctx * amask)
    acc[4] = acc[4] + jnp.sum(sv * cmt)
    acc[5] = acc[5] + jnp.sum(cmt * amask)

    @pl.when(i == n_blocks - 1)
    def _final():
        mean_ref[...] = jnp.full((1, 1), acc[0] / (acc[1] + 1e-8), jnp.float32)
        ctx_ref[...] = jnp.full((1, 1), acc[2] / (acc[3] + 1e-8), jnp.float32)
        cmt_ref[...] = jnp.full((1, 1), acc[4] / (acc[5] + 1e-8), jnp.float32)


def _tc_mul(emb2d, scores2d, tt2d, attn2d, blk):
    n, d = emb2d.shape
    n_blocks = n // blk
    row_spec = pl.BlockSpec((blk, 1), lambda i: (i, 0))
    scalar_spec = pl.BlockSpec((1, 1), lambda i: (0, 0))
    return pl.pallas_call(
        functools.partial(_tc_mul_body, n_blocks),
        grid=(n_blocks,),
        in_specs=[
            pl.BlockSpec((blk, d), lambda i: (i, 0)),
            row_spec,
            row_spec,
            row_spec,
        ],
        out_specs=[
            pl.BlockSpec((blk, d), lambda i: (i, 0)),
            scalar_spec,
            scalar_spec,
            scalar_spec,
        ],
        out_shape=[
            jax.ShapeDtypeStruct((n, d), jnp.float32),
            jax.ShapeDtypeStruct((1, 1), jnp.float32),
            jax.ShapeDtypeStruct((1, 1), jnp.float32),
            jax.ShapeDtypeStruct((1, 1), jnp.float32),
        ],
        scratch_shapes=[pltpu.SMEM((8,), jnp.float32)],
    )(emb2d, scores2d, tt2d, attn2d)


def kernel(embeddings, input_ids, token_type_ids, attention_mask,
           special_token_ids, tfidf_scores, default_score):
    b, l, d = embeddings.shape
    n = b * l

    ids_flat = input_ids.reshape(n).astype(jnp.int32)
    attn_flat = attention_mask.reshape(n).astype(jnp.float32)
    sp = special_token_ids.astype(jnp.int32)
    specials = [jnp.full((_LANES,), sp[k], jnp.int32) for k in range(4)]
    dflt_vec = jnp.full((_LANES,), default_score, jnp.float32)

    scores = _sc_scores(ids_flat, attn_flat,
                        tfidf_scores.astype(jnp.float32), specials, dflt_vec)

    emb2d = embeddings.reshape(n, d)
    sc2d = scores.reshape(n, 1)
    tt2d = token_type_ids.reshape(n, 1).astype(jnp.int32)
    attn2d = attn_flat.reshape(n, 1)

    masked, mean_v, ctx_v, cmt_v = _tc_mul(emb2d, sc2d, tt2d, attn2d, blk=2048)

    return (
        masked.reshape(b, l, d),
        scores.reshape(b, l, 1),
        mean_v[0, 0],
        ctx_v[0, 0],
        cmt_v[0, 0],
    )


# speedup vs baseline: 1.2728x; 1.2039x over previous
import functools

import jax
import jax.numpy as jnp
from jax import lax
from jax.experimental import pallas as pl
from jax.experimental.pallas import tpu as pltpu
from jax.experimental.pallas import tpu_sc as plsc

_NUM_CORES = 2
_NUM_SUBCORES = 16
_NW = _NUM_CORES * _NUM_SUBCORES
_LANES = 16
_CHUNK = 128
_NSUMS = 6


def _sc_scores_body(ids_hbm, attn_hbm, tt_hbm, tfidf_hbm, spec_hbm, dflt_hbm,
                    out_hbm, part_hbm,
                    idx_v, attn_v, tt_v, vals_v, spec_v, dflt_v, pbuf_v,
                    gsem, hsem):
    n_tok = ids_hbm.shape[0]
    per_w = n_tok // _NW
    n_chunks = per_w // _CHUNK
    wid = lax.axis_index("s") * _NUM_CORES + lax.axis_index("c")
    base = wid * per_w

    pltpu.sync_copy(ids_hbm.at[pl.ds(base, per_w)], idx_v)
    gathers = [
        pltpu.async_copy(
            tfidf_hbm.at[idx_v.at[pl.ds(j * _CHUNK, _CHUNK)]],
            vals_v.at[pl.ds(j * _CHUNK, _CHUNK)],
            gsem,
        )
        for j in range(n_chunks)
    ]
    others = [
        pltpu.async_copy(attn_hbm.at[pl.ds(base, per_w)], attn_v, hsem),
        pltpu.async_copy(tt_hbm.at[pl.ds(base, per_w)], tt_v, hsem),
        pltpu.async_copy(spec_hbm, spec_v, hsem),
        pltpu.async_copy(dflt_hbm, dflt_v, hsem),
    ]
    for c in gathers:
        c.wait()
    for c in others:
        c.wait()

    s0 = spec_v[pl.ds(0, _LANES)]
    s1 = spec_v[pl.ds(_LANES, _LANES)]
    s2 = spec_v[pl.ds(2 * _LANES, _LANES)]
    s3 = spec_v[pl.ds(3 * _LANES, _LANES)]
    dflt = dflt_v[...]
    one = jnp.ones((_LANES,), jnp.float32)
    zero = jnp.zeros((_LANES,), jnp.float32)
    acc = [zero] * _NSUMS
    for t in range(per_w // _LANES):
        sl = pl.ds(t * _LANES, _LANES)
        tok = idx_v[sl]
        val = vals_v[sl]
        amask = attn_v[sl]
        tt = tt_v[sl]
        is_special = (tok == s0) | (tok == s1) | (tok == s2) | (tok == s3)
        score = jnp.where(is_special, dflt, val) * amask
        vals_v[sl] = score
        ctx = jnp.where(tt == 1, one, zero)
        cmt = jnp.where(tt == 0, one, zero)
        acc[0] = acc[0] + score
        acc[1] = acc[1] + amask
        acc[2] = acc[2] + score * ctx
        acc[3] = acc[3] + ctx * amask
        acc[4] = acc[4] + score * cmt
        acc[5] = acc[5] + cmt * amask

    for k in range(_NSUMS):
        pbuf_v[pl.ds(k * _LANES, _LANES)] = acc[k]
    pbuf_v[pl.ds(_NSUMS * _LANES, _LANES)] = zero
    pbuf_v[pl.ds((_NSUMS + 1) * _LANES, _LANES)] = zero

    pltpu.sync_copy(vals_v, out_hbm.at[pl.ds(base, per_w)])
    pltpu.sync_copy(pbuf_v, part_hbm.at[pl.ds(wid * 128, 128)])


def _sc_scores(ids_flat, attn_flat, tt_flat, tfidf, spec_vec, dflt_vec):
    n_tok = ids_flat.shape[0]
    per_w = n_tok // _NW
    call = functools.partial(
        pl.kernel,
        mesh=plsc.VectorSubcoreMesh(core_axis_name="c", subcore_axis_name="s"),
        out_type=[
            jax.ShapeDtypeStruct((n_tok,), jnp.float32),
            jax.ShapeDtypeStruct((_NW * 128,), jnp.float32),
        ],
        scratch_types=[
            pltpu.VMEM((per_w,), jnp.int32),
            pltpu.VMEM((per_w,), jnp.float32),
            pltpu.VMEM((per_w,), jnp.int32),
            pltpu.VMEM((per_w,), jnp.float32),
            pltpu.VMEM((4 * _LANES,), jnp.int32),
            pltpu.VMEM((_LANES,), jnp.float32),
            pltpu.VMEM((128,), jnp.float32),
            pltpu.SemaphoreType.DMA,
            pltpu.SemaphoreType.DMA,
        ],
    )(_sc_scores_body)
    return call(ids_flat, attn_flat, tt_flat, tfidf, spec_vec, dflt_vec)


def _tc_mul_body(n_blocks, emb_ref, sc_ref, part_ref, out_ref,
                 mean_ref, ctx_ref, cmt_ref):
    i = pl.program_id(0)
    out_ref[...] = emb_ref[...] * sc_ref[...]

    @pl.when(i == n_blocks - 1)
    def _final():
        p = part_ref[...]
        lane = lax.broadcasted_iota(jnp.int32, p.shape, 1) // _LANES
        sums = [jnp.sum(jnp.where(lane == k, p, 0.0)) for k in range(_NSUMS)]
        mean_ref[...] = jnp.full((1, 1), sums[0] / (sums[1] + 1e-8),
                                 jnp.float32)
        ctx_ref[...] = jnp.full((1, 1), sums[2] / (sums[3] + 1e-8),
                                jnp.float32)
        cmt_ref[...] = jnp.full((1, 1), sums[4] / (sums[5] + 1e-8),
                                jnp.float32)


def _tc_mul(emb2d, scores2d, partials, blk):
    n, d = emb2d.shape
    n_blocks = n // blk
    scalar_spec = pl.BlockSpec((1, 1), lambda i: (0, 0))
    return pl.pallas_call(
        functools.partial(_tc_mul_body, n_blocks),
        grid=(n_blocks,),
        in_specs=[
            pl.BlockSpec((blk, d), lambda i: (i, 0)),
            pl.BlockSpec((blk, 1), lambda i: (i, 0)),
            pl.BlockSpec((_NW, 128), lambda i: (0, 0)),
        ],
        out_specs=[
            pl.BlockSpec((blk, d), lambda i: (i, 0)),
            scalar_spec,
            scalar_spec,
            scalar_spec,
        ],
        out_shape=[
            jax.ShapeDtypeStruct((n, d), jnp.float32),
            jax.ShapeDtypeStruct((1, 1), jnp.float32),
            jax.ShapeDtypeStruct((1, 1), jnp.float32),
            jax.ShapeDtypeStruct((1, 1), jnp.float32),
        ],
    )(emb2d, scores2d, partials)


def kernel(embeddings, input_ids, token_type_ids, attention_mask,
           special_token_ids, tfidf_scores, default_score):
    b, l, d = embeddings.shape
    n = b * l

    ids_flat = input_ids.reshape(n).astype(jnp.int32)
    attn_flat = attention_mask.reshape(n).astype(jnp.float32)
    tt_flat = token_type_ids.reshape(n).astype(jnp.int32)
    sp = special_token_ids.astype(jnp.int32)
    spec_vec = jnp.repeat(sp, _LANES)
    dflt_vec = jnp.full((_LANES,), default_score, jnp.float32)

    scores, partials = _sc_scores(ids_flat, attn_flat, tt_flat,
                                  tfidf_scores.astype(jnp.float32),
                                  spec_vec, dflt_vec)

    emb2d = embeddings.reshape(n, d)
    sc2d = scores.reshape(n, 1)
    part2d = partials.reshape(_NW, 128)

    masked, mean_v, ctx_v, cmt_v = _tc_mul(emb2d, sc2d, part2d, blk=2048)

    return (
        masked.reshape(b, l, d),
        scores.reshape(b, l, 1),
        mean_v[0, 0],
        ctx_v[0, 0],
        cmt_v[0, 0],
    )
